# baseline (speedup 1.0000x reference)
import functools

import jax
import jax.numpy as jnp
from jax import lax
from jax.experimental import pallas as pl
from jax.experimental.pallas import tpu as pltpu
from jax.experimental.pallas import tpu_sc as plsc

VOCAB = 100000
HIDDEN = 768
MAX_POS = 2048
EPS = 1e-12
B, S = 4, 2048
NTOK = B * S

L = 16
NC, NS = 2, 16
NW = NC * NS
TPW = NTOK // NW
C = 32
NCHUNK = TPW // C
HV = HIDDEN // L


def _lane_allsum(x):
    lanes = lax.iota(jnp.int32, L)
    dnums = lax.GatherDimensionNumbers(
        offset_dims=(), collapsed_slice_dims=(0,), start_index_map=(0,))
    for k in (8, 4, 2, 1):
        perm = (lanes ^ k)[:, None]
        x = x + lax.gather(x, perm, dnums, (1,),
                           mode=lax.GatherScatterMode.PROMISE_IN_BOUNDS)
    return x


def _rsqrt(v):
    vi = lax.bitcast_convert_type(v, jnp.int32)
    yi = jnp.int32(0x5F3759DF) - (vi >> 1)
    y = lax.bitcast_convert_type(yi, jnp.float32)
    for _ in range(3):
        y = y * (1.5 - 0.5 * v * y * y)
    return y


def _body(ids_hbm, tt_hbm, word_hbm, pos_hbm, type_hbm, lnw_hbm, lnb_hbm,
          out_hbm, idx_v, tti_v, wbuf, pbuf, tbuf, lnw_v, lnb_v, sem_w,
          sem_t):
    wid = lax.axis_index("s") * NC + lax.axis_index("c")
    pltpu.sync_copy(lnw_hbm, lnw_v)
    pltpu.sync_copy(lnb_hbm, lnb_v)

    def chunk_body(c, carry):
        base = wid * TPW + c * C
        s_base = lax.rem(base, S)
        pltpu.sync_copy(ids_hbm.at[pl.ds(base, C)], idx_v)
        pltpu.sync_copy(tt_hbm.at[pl.ds(base, C)], tti_v)
        cp_w = pltpu.async_copy(word_hbm.at[idx_v], wbuf, sem_w)
        cp_t = pltpu.async_copy(type_hbm.at[tti_v], tbuf, sem_t)
        pltpu.sync_copy(pos_hbm.at[pl.ds(s_base, C)], pbuf)
        cp_w.wait()
        cp_t.wait()

        def tok_body(i, carry2):
            acc_s = jnp.zeros((L,), jnp.float32)
            acc_q = jnp.zeros((L,), jnp.float32)
            for h in range(HV):
                sl = pl.ds(h * L, L)
                x = wbuf[i, sl] + pbuf[i, sl] + tbuf[i, sl]
                wbuf[i, sl] = x
                acc_s = acc_s + x
                acc_q = acc_q + x * x
            mean_v = _lane_allsum(acc_s) * (1.0 / HIDDEN)
            var_v = _lane_allsum(acc_q) * (1.0 / HIDDEN) - mean_v * mean_v
            rinv = _rsqrt(var_v + EPS)
            for h in range(HV):
                sl = pl.ds(h * L, L)
                x = wbuf[i, sl]
                wbuf[i, sl] = (x - mean_v) * rinv * lnw_v[sl] + lnb_v[sl]
            return carry2

        lax.fori_loop(0, C, tok_body, 0)
        pltpu.sync_copy(wbuf, out_hbm.at[pl.ds(base, C)])
        return carry

    lax.fori_loop(0, NCHUNK, chunk_body, 0)


@jax.jit
def _emb_ln(ids_flat, tt_flat, word_emb, pos_emb, type_emb, ln_w, ln_b):
    mesh = plsc.VectorSubcoreMesh(core_axis_name="c", subcore_axis_name="s")
    k = functools.partial(
        pl.kernel,
        out_type=jax.ShapeDtypeStruct((NTOK, HIDDEN), jnp.float32),
        mesh=mesh,
        scratch_types=[
            pltpu.VMEM((C,), jnp.int32),
            pltpu.VMEM((C,), jnp.int32),
            pltpu.VMEM((C, HIDDEN), jnp.float32),
            pltpu.VMEM((C, HIDDEN), jnp.float32),
            pltpu.VMEM((C, HIDDEN), jnp.float32),
            pltpu.VMEM((HIDDEN,), jnp.float32),
            pltpu.VMEM((HIDDEN,), jnp.float32),
            pltpu.SemaphoreType.DMA,
            pltpu.SemaphoreType.DMA,
        ],
    )(_body)
    return k(ids_flat, tt_flat, word_emb, pos_emb, type_emb, ln_w, ln_b)


def kernel(input_ids, token_type_ids, word_emb, pos_emb, type_emb, ln_w,
           ln_b):
    out = _emb_ln(input_ids.reshape(-1), token_type_ids.reshape(-1),
                  word_emb, pos_emb, type_emb, ln_w, ln_b)
    return out.reshape(B, S, HIDDEN)
